# Initial kernel scaffold; baseline (speedup 1.0000x reference)
import jax
import jax.numpy as jnp
from jax.experimental import pallas as pl


def kernel(x, edge_index, edge_attr, W1, b1, W2, b2, W3, b3):
    raise NotImplementedError("write your pallas kernel here")


# speedup vs baseline: 15.0315x; 15.0315x over previous
import functools

import jax
import jax.numpy as jnp
from jax import lax
from jax.experimental import pallas as pl
from jax.experimental.pallas import tpu as pltpu
from jax.experimental.pallas import tpu_sc as plsc

N = 10000
NPAD = 10240
E = 320000
D = 128
H = 128
C = 64

RPS = 125
EDGE_ROWS = E // RPS
N_TILES = 32
RPT = EDGE_ROWS // N_TILES
OPT = NPAD // 16

MB = 1000


def _mesh():
    return plsc.VectorSubcoreMesh(core_axis_name="c", subcore_axis_name="s")


@functools.cache
def _make_deg():
    @functools.partial(
        pl.kernel,
        out_type=jax.ShapeDtypeStruct((2, NPAD), jnp.float32),
        mesh=_mesh(),
        scratch_types=[
            pltpu.VMEM((RPT, RPS), jnp.int32),
            pltpu.VMEM((128,), jnp.float32),
            pltpu.VMEM((OPT,), jnp.float32),
            pltpu.VMEM_SHARED((NPAD,), jnp.float32),
        ],
    )
    def _deg_sc(dst_hbm, out_hbm, dst_v, ones_v, z_v, acc):
        c = lax.axis_index("c")
        s = lax.axis_index("s")
        base_row = (c * 16 + s) * RPT
        pltpu.sync_copy(dst_hbm.at[pl.ds(base_row, RPT)], dst_v)

        def fill_ones(i, carry):
            ones_v[pl.ds(i * 16, 16)] = jnp.ones((16,), jnp.float32)
            return carry
        lax.fori_loop(0, 8, fill_ones, 0)

        def fill_zeros(i, carry):
            z_v[pl.ds(i * 16, 16)] = jnp.zeros((16,), jnp.float32)
            return carry
        lax.fori_loop(0, OPT // 16, fill_zeros, 0)

        pltpu.sync_copy(z_v, acc.at[pl.ds(s * OPT, OPT)])
        plsc.subcore_barrier()

        def body(j, carry):
            pltpu.sync_copy(ones_v.at[pl.ds(0, RPS)], acc.at[dst_v.at[j]], add=True)
            return carry
        lax.fori_loop(0, RPT, body, 0)

        plsc.subcore_barrier()
        pltpu.sync_copy(acc.at[pl.ds(s * OPT, OPT)],
                        out_hbm.at[c, pl.ds(s * OPT, OPT)])
    return _deg_sc


@functools.cache
def _make_segsum(Hw):
    @functools.partial(
        pl.kernel,
        out_type=jax.ShapeDtypeStruct((2, NPAD, Hw), jnp.float32),
        mesh=_mesh(),
        scratch_types=[
            pltpu.VMEM((RPT, RPS), jnp.int32),
            pltpu.VMEM((RPT, RPS), jnp.int32),
            pltpu.VMEM((RPS, Hw), jnp.float32),
            pltpu.VMEM_SHARED((NPAD, Hw), jnp.float32),
            pltpu.SemaphoreType.DMA,
        ],
    )
    def seg(feat_hbm, src_hbm, dst_hbm, out_hbm, src_v, dst_v, rows_v, acc, sem):
        c = lax.axis_index("c")
        s = lax.axis_index("s")
        base_row = (c * 16 + s) * RPT
        pltpu.sync_copy(src_hbm.at[pl.ds(base_row, RPT)], src_v)
        pltpu.sync_copy(dst_hbm.at[pl.ds(base_row, RPT)], dst_v)

        def fzrow(r, carry):
            for k in range(Hw // 16):
                rows_v[r, pl.ds(k * 16, 16)] = jnp.zeros((16,), jnp.float32)
            return carry
        lax.fori_loop(0, RPT, fzrow, 0)

        for t in range(OPT // RPT):
            pltpu.sync_copy(rows_v.at[pl.ds(0, RPT)],
                            acc.at[pl.ds(s * OPT + t * RPT, RPT)])
        plsc.subcore_barrier()

        def body(j, carry):
            pltpu.async_copy(feat_hbm.at[src_v.at[j]], rows_v, sem).wait()
            pltpu.sync_copy(rows_v, acc.at[dst_v.at[j]], add=True)
            return carry
        lax.fori_loop(0, RPT, body, 0)

        plsc.subcore_barrier()
        pltpu.sync_copy(acc.at[pl.ds(s * OPT, OPT)],
                        out_hbm.at[c, pl.ds(s * OPT, OPT)])
    return seg


def _dsq(deg_ref):
    deg = deg_ref[:, 0] + deg_ref[:, 1]
    return lax.rsqrt(jnp.maximum(deg, 1.0))


def _k1_body(x_ref, w_ref, deg_ref, o_ref):
    dsq = _dsq(deg_ref)
    o_ref[...] = jnp.dot(x_ref[...], w_ref[...],
                         preferred_element_type=jnp.float32) * dsq[:, None]


def _k1(x, W, degp):
    return pl.pallas_call(
        _k1_body,
        grid=(N // MB,),
        in_specs=[
            pl.BlockSpec((MB, D), lambda i: (i, 0)),
            pl.BlockSpec((D, H), lambda i: (0, 0)),
            pl.BlockSpec((MB, 2), lambda i: (i, 0)),
        ],
        out_specs=pl.BlockSpec((MB, H), lambda i: (i, 0)),
        out_shape=jax.ShapeDtypeStruct((N, H), jnp.float32),
    )(x, W, degp)


def _k2_body(rp_ref, deg_ref, b_ref, w_ref, o_ref):
    dsq = _dsq(deg_ref)
    agg = rp_ref[0] + rp_ref[1]
    h = jnp.maximum(agg * dsq[:, None] + b_ref[...], 0.0)
    o_ref[...] = jnp.dot(h, w_ref[...],
                         preferred_element_type=jnp.float32) * dsq[:, None]


def _k2(rp, degp, b, W):
    Wout = W.shape[1]
    return pl.pallas_call(
        _k2_body,
        grid=(N // MB,),
        in_specs=[
            pl.BlockSpec((2, MB, H), lambda i: (0, i, 0)),
            pl.BlockSpec((MB, 2), lambda i: (i, 0)),
            pl.BlockSpec((1, H), lambda i: (0, 0)),
            pl.BlockSpec((H, Wout), lambda i: (0, 0)),
        ],
        out_specs=pl.BlockSpec((MB, Wout), lambda i: (i, 0)),
        out_shape=jax.ShapeDtypeStruct((N, Wout), jnp.float32),
    )(rp, degp, b.reshape(1, H), W)


def _k4_body(rp_ref, deg_ref, b_ref, o_ref):
    dsq = _dsq(deg_ref)
    y = (rp_ref[0, :, :C] + rp_ref[1, :, :C]) * dsq[:, None] + b_ref[...]
    m = jnp.max(y, axis=1, keepdims=True)
    lse = jnp.log(jnp.sum(jnp.exp(y - m), axis=1, keepdims=True)) + m
    o_ref[...] = y - lse


def _k4(rp, degp, b):
    return pl.pallas_call(
        _k4_body,
        grid=(N // MB,),
        in_specs=[
            pl.BlockSpec((2, MB, H), lambda i: (0, i, 0)),
            pl.BlockSpec((MB, 2), lambda i: (i, 0)),
            pl.BlockSpec((1, C), lambda i: (0, 0)),
        ],
        out_specs=pl.BlockSpec((MB, C), lambda i: (i, 0)),
        out_shape=jax.ShapeDtypeStruct((N, C), jnp.float32),
    )(rp, degp, b.reshape(1, C))


def kernel(x, edge_index, edge_attr, W1, b1, W2, b2, W3, b3):
    src = edge_index[0].reshape(EDGE_ROWS, RPS)
    dst = edge_index[1].reshape(EDGE_ROWS, RPS)

    degp = _make_deg()(dst)[:, :N].T
    _seg128 = _make_segsum(H)
    s1 = _k1(x, W1, degp)
    r1 = _seg128(s1, src, dst)[:, :N]
    s2 = _k2(r1, degp, b1, W2)
    r2 = _seg128(s2, src, dst)[:, :N]
    W3p = jnp.zeros((H, H), W3.dtype).at[:, :C].set(W3)
    s3 = _k2(r2, degp, b2, W3p)
    r3 = _seg128(s3, src, dst)[:, :N]
    return _k4(r3, degp, b3)


# speedup vs baseline: 18.9994x; 1.2640x over previous
import functools

import jax
import jax.numpy as jnp
from jax import lax
from jax.experimental import pallas as pl
from jax.experimental.pallas import tpu as pltpu
from jax.experimental.pallas import tpu_sc as plsc

N = 10000
NPAD = 10240
E = 320000
D = 128
H = 128
C = 64

RPS = 125
EDGE_ROWS = E // RPS
N_TILES = 32
RPT = EDGE_ROWS // N_TILES
OPT = NPAD // 16

MB = 1000


def _mesh():
    return plsc.VectorSubcoreMesh(core_axis_name="c", subcore_axis_name="s")


@functools.cache
def _make_deg():
    @functools.partial(
        pl.kernel,
        out_type=jax.ShapeDtypeStruct((2, NPAD), jnp.float32),
        mesh=_mesh(),
        scratch_types=[
            pltpu.VMEM((RPT, RPS), jnp.int32),
            pltpu.VMEM((128,), jnp.float32),
            pltpu.VMEM((OPT,), jnp.float32),
            pltpu.VMEM_SHARED((NPAD,), jnp.float32),
        ],
    )
    def _deg_sc(dst_hbm, out_hbm, dst_v, ones_v, z_v, acc):
        c = lax.axis_index("c")
        s = lax.axis_index("s")
        base_row = (c * 16 + s) * RPT
        pltpu.sync_copy(dst_hbm.at[pl.ds(base_row, RPT)], dst_v)

        def fill_ones(i, carry):
            ones_v[pl.ds(i * 16, 16)] = jnp.ones((16,), jnp.float32)
            return carry
        lax.fori_loop(0, 8, fill_ones, 0)

        def fill_zeros(i, carry):
            z_v[pl.ds(i * 16, 16)] = jnp.zeros((16,), jnp.float32)
            return carry
        lax.fori_loop(0, OPT // 16, fill_zeros, 0)

        pltpu.sync_copy(z_v, acc.at[pl.ds(s * OPT, OPT)])
        plsc.subcore_barrier()

        def body(j, carry):
            pltpu.sync_copy(ones_v.at[pl.ds(0, RPS)], acc.at[dst_v.at[j]], add=True)
            return carry
        lax.fori_loop(0, RPT, body, 0)

        plsc.subcore_barrier()
        pltpu.sync_copy(acc.at[pl.ds(s * OPT, OPT)],
                        out_hbm.at[c, pl.ds(s * OPT, OPT)])
    return _deg_sc


HPT = RPT // 2


@functools.cache
def _make_segsum(Hw):
    @functools.partial(
        pl.kernel,
        out_type=jax.ShapeDtypeStruct((2, NPAD, Hw), jnp.float32),
        mesh=_mesh(),
        scratch_types=[
            pltpu.VMEM((HPT, RPS), jnp.int32),
            pltpu.VMEM((HPT, RPS), jnp.int32),
            pltpu.VMEM((RPS, Hw), jnp.float32),
            pltpu.VMEM((RPS, Hw), jnp.float32),
            pltpu.VMEM_SHARED((NPAD, Hw), jnp.float32),
            pltpu.SemaphoreType.DMA,
            pltpu.SemaphoreType.DMA,
            pltpu.SemaphoreType.DMA,
            pltpu.SemaphoreType.DMA,
        ],
    )
    def seg(feat_hbm, src_hbm, dst_hbm, out_hbm, src_v, dst_v, buf0, buf1,
            acc_ref, gs0, gs1, ss0, ss1):
        c = lax.axis_index("c")
        s = lax.axis_index("s")
        base_row = (c * 16 + s) * RPT
        def fzrow(r, carry):
            for k in range(Hw // 16):
                buf0[r, pl.ds(k * 16, 16)] = jnp.zeros((16,), jnp.float32)
            return carry
        lax.fori_loop(0, RPT, fzrow, 0)
        for t in range(OPT // RPT):
            pltpu.sync_copy(buf0.at[pl.ds(0, RPT)],
                            acc_ref.at[pl.ds(s * OPT + t * RPT, RPT)])
        plsc.subcore_barrier()

        for p in range(RPT // HPT):
            pltpu.sync_copy(src_hbm.at[pl.ds(base_row + p * HPT, HPT)], src_v)
            pltpu.sync_copy(dst_hbm.at[pl.ds(base_row + p * HPT, HPT)], dst_v)
            pltpu.async_copy(feat_hbm.at[src_v.at[0]], buf0, gs0)

            def body(i, carry):
                j0 = 2 * i
                pltpu.make_async_copy(feat_hbm.at[src_v.at[j0]], buf0, gs0).wait()
                pltpu.async_copy(feat_hbm.at[src_v.at[j0 + 1]], buf1, gs1)
                pltpu.async_copy(buf0, acc_ref.at[dst_v.at[j0]], ss0, add=True)
                pltpu.make_async_copy(feat_hbm.at[src_v.at[j0 + 1]], buf1, gs1).wait()
                pltpu.make_async_copy(buf0, acc_ref.at[dst_v.at[j0]], ss0).wait()

                @pl.when(i < HPT // 2 - 1)
                def _():
                    pltpu.async_copy(feat_hbm.at[src_v.at[j0 + 2]], buf0, gs0)
                pltpu.async_copy(buf1, acc_ref.at[dst_v.at[j0 + 1]], ss1, add=True)
                pltpu.make_async_copy(buf1, acc_ref.at[dst_v.at[j0 + 1]], ss1).wait()
                return carry
            lax.fori_loop(0, HPT // 2, body, 0)

        plsc.subcore_barrier()
        pltpu.sync_copy(acc_ref.at[pl.ds(s * OPT, OPT)],
                        out_hbm.at[c, pl.ds(s * OPT, OPT)])
    return seg


def _dsq(deg_ref):
    deg = deg_ref[:, 0] + deg_ref[:, 1]
    return lax.rsqrt(jnp.maximum(deg, 1.0))


def _k1_body(x_ref, w_ref, deg_ref, o_ref):
    dsq = _dsq(deg_ref)
    o_ref[...] = jnp.dot(x_ref[...], w_ref[...],
                         preferred_element_type=jnp.float32) * dsq[:, None]


def _k1(x, W, degp):
    return pl.pallas_call(
        _k1_body,
        grid=(N // MB,),
        in_specs=[
            pl.BlockSpec((MB, D), lambda i: (i, 0)),
            pl.BlockSpec((D, H), lambda i: (0, 0)),
            pl.BlockSpec((MB, 2), lambda i: (i, 0)),
        ],
        out_specs=pl.BlockSpec((MB, H), lambda i: (i, 0)),
        out_shape=jax.ShapeDtypeStruct((N, H), jnp.float32),
    )(x, W, degp)


def _k2_body(rp_ref, deg_ref, b_ref, w_ref, o_ref):
    dsq = _dsq(deg_ref)
    agg = rp_ref[0] + rp_ref[1]
    h = jnp.maximum(agg * dsq[:, None] + b_ref[...], 0.0)
    o_ref[...] = jnp.dot(h, w_ref[...],
                         preferred_element_type=jnp.float32) * dsq[:, None]


def _k2(rp, degp, b, W):
    Wout = W.shape[1]
    return pl.pallas_call(
        _k2_body,
        grid=(N // MB,),
        in_specs=[
            pl.BlockSpec((2, MB, H), lambda i: (0, i, 0)),
            pl.BlockSpec((MB, 2), lambda i: (i, 0)),
            pl.BlockSpec((1, H), lambda i: (0, 0)),
            pl.BlockSpec((H, Wout), lambda i: (0, 0)),
        ],
        out_specs=pl.BlockSpec((MB, Wout), lambda i: (i, 0)),
        out_shape=jax.ShapeDtypeStruct((N, Wout), jnp.float32),
    )(rp, degp, b.reshape(1, H), W)


def _k4_body(rp_ref, deg_ref, b_ref, o_ref):
    dsq = _dsq(deg_ref)
    y = (rp_ref[0, :, :C] + rp_ref[1, :, :C]) * dsq[:, None] + b_ref[...]
    m = jnp.max(y, axis=1, keepdims=True)
    lse = jnp.log(jnp.sum(jnp.exp(y - m), axis=1, keepdims=True)) + m
    o_ref[...] = y - lse


def _k4(rp, degp, b):
    return pl.pallas_call(
        _k4_body,
        grid=(N // MB,),
        in_specs=[
            pl.BlockSpec((2, MB, H), lambda i: (0, i, 0)),
            pl.BlockSpec((MB, 2), lambda i: (i, 0)),
            pl.BlockSpec((1, C), lambda i: (0, 0)),
        ],
        out_specs=pl.BlockSpec((MB, C), lambda i: (i, 0)),
        out_shape=jax.ShapeDtypeStruct((N, C), jnp.float32),
    )(rp, degp, b.reshape(1, C))


def kernel(x, edge_index, edge_attr, W1, b1, W2, b2, W3, b3):
    src = edge_index[0].reshape(EDGE_ROWS, RPS)
    dst = edge_index[1].reshape(EDGE_ROWS, RPS)

    degp = _make_deg()(dst)[:, :N].T
    _seg128 = _make_segsum(H)
    s1 = _k1(x, W1, degp)
    r1 = _seg128(s1, src, dst)[:, :N]
    s2 = _k2(r1, degp, b1, W2)
    r2 = _seg128(s2, src, dst)[:, :N]
    W3p = jnp.zeros((H, H), W3.dtype).at[:, :C].set(W3)
    s3 = _k2(r2, degp, b2, W3p)
    r3 = _seg128(s3, src, dst)[:, :N]
    return _k4(r3, degp, b3)
